# baseline (speedup 1.0000x reference)
import jax
import jax.numpy as jnp
from jax import lax
from jax.experimental import pallas as pl
from jax.experimental.pallas import tpu as pltpu
from jax.experimental.pallas import tpu_sc as plsc

BATCH = 4
SEQ = 2048
D = 768
LANES = 16
NUM_WORKERS = 32
P = SEQ // NUM_WORKERS
CP = 8
NCHUNK = P // CP
NB = 4
LOOKAHEAD = 3
ROWS = BATCH * CP
VREGS_PER_ROW = D // LANES


def _body(ids_hbm, wte_hbm, wpe_hbm, out_hbm,
          idx_v, gbufs, wbufs, gsem, wsem, ssem):
    wid = lax.axis_index("s") * 2 + lax.axis_index("c")
    pos0 = wid * P

    pltpu.sync_copy(ids_hbm.at[wid], idx_v)

    def wdesc(k, st):
        return pltpu.make_async_copy(
            wpe_hbm.at[pl.ds(pos0 + k * CP, CP)], wbufs.at[st], wsem.at[st])

    def gdesc(k, st):
        return pltpu.make_async_copy(
            wte_hbm.at[idx_v.at[k]], gbufs.at[st], gsem.at[st])

    def sdesc(k, st, b):
        return pltpu.make_async_copy(
            gbufs.at[st, pl.ds(b * CP, CP)],
            out_hbm.at[b, pl.ds(pos0 + k * CP, CP)], ssem.at[st])

    def issue_loads(k, st):
        wdesc(k, st).start()
        gdesc(k, st).start()

    for k in range(min(LOOKAHEAD, NCHUNK)):
        issue_loads(k, k % NB)

    def chunk(k, carry):
        st = jnp.bitwise_and(k, NB - 1)
        wdesc(k, st).wait()
        gdesc(k, st).wait()

        def add_row(r, c):
            for j in range(VREGS_PER_ROW):
                sl = pl.ds(j * LANES, LANES)
                v = wbufs[st, r, sl]
                for b in range(BATCH):
                    plsc.addupdate(gbufs.at[st, b * CP + r, sl], v)
            return c

        lax.fori_loop(0, CP, add_row, 0)

        for b in range(BATCH):
            sdesc(k, st, b).start()

        nk = k + LOOKAHEAD

        @pl.when(nk < NCHUNK)
        def _():
            pk = k - 1

            @pl.when(pk >= 0)
            def _():
                pst = jnp.bitwise_and(pk, NB - 1)
                for b in range(BATCH):
                    sdesc(pk, pst, b).wait()

            issue_loads(nk, jnp.bitwise_and(nk, NB - 1))

        return carry

    lax.fori_loop(0, NCHUNK, chunk, 0)

    for k in range(max(0, NCHUNK - NB), NCHUNK):
        for b in range(BATCH):
            sdesc(k, k % NB, b).wait()


@jax.jit
def _embed(input_ids, wte, wpe):
    ids = input_ids.reshape(BATCH, NUM_WORKERS, NCHUNK, CP)
    ids = ids.transpose(1, 2, 0, 3).reshape(NUM_WORKERS, NCHUNK, ROWS)

    mesh = plsc.VectorSubcoreMesh(core_axis_name="c", subcore_axis_name="s")
    return pl.kernel(
        _body,
        out_type=jax.ShapeDtypeStruct((BATCH, SEQ, D), jnp.float32),
        mesh=mesh,
        scratch_types=[
            pltpu.VMEM((NCHUNK, ROWS), jnp.int32),
            pltpu.VMEM((NB, ROWS, D), jnp.float32),
            pltpu.VMEM((NB, CP, D), jnp.float32),
            pltpu.SemaphoreType.DMA((NB,)),
            pltpu.SemaphoreType.DMA((NB,)),
            pltpu.SemaphoreType.DMA((NB,)),
        ],
    )(ids, wte, wpe)


def kernel(input_ids, wte, wpe):
    return _embed(input_ids, wte, wpe)


# speedup vs baseline: 1.0224x; 1.0224x over previous
import jax
import jax.numpy as jnp
from jax import lax
from jax.experimental import pallas as pl
from jax.experimental.pallas import tpu as pltpu
from jax.experimental.pallas import tpu_sc as plsc

BATCH = 4
SEQ = 2048
D = 768
LANES = 16
NUM_WORKERS = 32
P = SEQ // NUM_WORKERS
CP = 8
NCHUNK = P // CP
NB = 4
LOOKAHEAD = 3
ROWS = BATCH * CP
VREGS_PER_ROW = D // LANES


def _body(ids_hbm, wte_hbm, wpe_hbm, out_hbm,
          idx_v, gbufs, wbufs, gsem, wsem, ssem):
    wid = lax.axis_index("s") * 2 + lax.axis_index("c")
    pos0 = wid * P

    pltpu.sync_copy(ids_hbm.at[wid], idx_v)

    def wdesc(k, st):
        return pltpu.make_async_copy(
            wpe_hbm.at[pl.ds(pos0 + k * CP, CP)], wbufs.at[st], wsem.at[st])

    def gdesc(k, st):
        return pltpu.make_async_copy(
            wte_hbm.at[idx_v.at[k]], gbufs.at[st], gsem.at[st])

    def sdesc(k, st, b):
        return pltpu.make_async_copy(
            gbufs.at[st, pl.ds(b * CP, CP)],
            out_hbm.at[b, pl.ds(pos0 + k * CP, CP)], ssem.at[st])

    def issue_loads(k, st):
        wdesc(k, st).start()
        gdesc(k, st).start()

    for k in range(min(LOOKAHEAD, NCHUNK)):
        issue_loads(k, k % NB)

    def chunk(k, carry):
        st = jnp.bitwise_and(k, NB - 1)
        wdesc(k, st).wait()
        gdesc(k, st).wait()

        def add_row(r, c):
            def add_group(jg, c2):
                for u in range(8):
                    sl = pl.ds((jg * 8 + u) * LANES, LANES)
                    v = wbufs[st, r, sl]
                    for b in range(BATCH):
                        plsc.addupdate(gbufs.at[st, b * CP + r, sl], v)
                return c2

            return lax.fori_loop(0, VREGS_PER_ROW // 8, add_group, c)

        lax.fori_loop(0, CP, add_row, 0)

        for b in range(BATCH):
            sdesc(k, st, b).start()

        nk = k + LOOKAHEAD

        @pl.when(nk < NCHUNK)
        def _():
            pk = k - 1

            @pl.when(pk >= 0)
            def _():
                pst = jnp.bitwise_and(pk, NB - 1)
                for b in range(BATCH):
                    sdesc(pk, pst, b).wait()

            issue_loads(nk, jnp.bitwise_and(nk, NB - 1))

        return carry

    lax.fori_loop(0, NCHUNK, chunk, 0)

    for k in range(max(0, NCHUNK - NB), NCHUNK):
        for b in range(BATCH):
            sdesc(k, k % NB, b).wait()


@jax.jit
def _embed(input_ids, wte, wpe):
    ids = input_ids.reshape(BATCH, NUM_WORKERS, NCHUNK, CP)
    ids = ids.transpose(1, 2, 0, 3).reshape(NUM_WORKERS, NCHUNK, ROWS)

    mesh = plsc.VectorSubcoreMesh(core_axis_name="c", subcore_axis_name="s")
    return pl.kernel(
        _body,
        out_type=jax.ShapeDtypeStruct((BATCH, SEQ, D), jnp.float32),
        mesh=mesh,
        scratch_types=[
            pltpu.VMEM((NCHUNK, ROWS), jnp.int32),
            pltpu.VMEM((NB, ROWS, D), jnp.float32),
            pltpu.VMEM((NB, CP, D), jnp.float32),
            pltpu.SemaphoreType.DMA((NB,)),
            pltpu.SemaphoreType.DMA((NB,)),
            pltpu.SemaphoreType.DMA((NB,)),
        ],
    )(ids, wte, wpe)


def kernel(input_ids, wte, wpe):
    return _embed(input_ids, wte, wpe)


# speedup vs baseline: 1.0244x; 1.0019x over previous
import jax
import jax.numpy as jnp
from jax import lax
from jax.experimental import pallas as pl
from jax.experimental.pallas import tpu as pltpu
from jax.experimental.pallas import tpu_sc as plsc

BATCH = 4
SEQ = 2048
D = 768
LANES = 16
NUM_WORKERS = 32
P = SEQ // NUM_WORKERS
CP = 8
NCHUNK = P // CP
NB = 4
LOOKAHEAD = 3
ROWS = BATCH * CP
VREGS_PER_ROW = D // LANES


def _body(ids_hbm, wte_hbm, wpe_hbm, out_hbm,
          idx_v, gbufs, wbufs, gsem, wsem, ssem, isem):
    wid = lax.axis_index("s") * 2 + lax.axis_index("c")
    pos0 = wid * P

    def wdesc(k, st):
        return pltpu.make_async_copy(
            wpe_hbm.at[pl.ds(pos0 + k * CP, CP)], wbufs.at[st], wsem.at[st])

    def gdesc(k, st, b):
        return pltpu.make_async_copy(
            wte_hbm.at[idx_v.at[b, pl.ds(k * CP, CP)]],
            gbufs.at[st, pl.ds(b * CP, CP)], gsem.at[st])

    def sdesc(k, st, b):
        return pltpu.make_async_copy(
            gbufs.at[st, pl.ds(b * CP, CP)],
            out_hbm.at[b, pl.ds(pos0 + k * CP, CP)], ssem.at[st])

    for k in range(min(LOOKAHEAD, NCHUNK)):
        wdesc(k, k % NB).start()
    idx_copies = [
        pltpu.make_async_copy(
            ids_hbm.at[b, pl.ds(pos0, P)], idx_v.at[b], isem)
        for b in range(BATCH)
    ]
    for h in idx_copies:
        h.start()
    for h in idx_copies:
        h.wait()
    for k in range(min(LOOKAHEAD, NCHUNK)):
        for b in range(BATCH):
            gdesc(k, k % NB, b).start()

    def chunk(k, carry):
        st = jnp.bitwise_and(k, NB - 1)
        wdesc(k, st).wait()
        for b in range(BATCH):
            gdesc(k, st, b).wait()

        def add_row(r, c):
            def add_group(jg, c2):
                for u in range(8):
                    sl = pl.ds((jg * 8 + u) * LANES, LANES)
                    v = wbufs[st, r, sl]
                    for b in range(BATCH):
                        plsc.addupdate(gbufs.at[st, b * CP + r, sl], v)
                return c2

            return lax.fori_loop(0, VREGS_PER_ROW // 8, add_group, c)

        lax.fori_loop(0, CP, add_row, 0)

        for b in range(BATCH):
            sdesc(k, st, b).start()

        nk = k + LOOKAHEAD

        @pl.when(nk < NCHUNK)
        def _():
            pk = k - 1

            @pl.when(pk >= 0)
            def _():
                pst = jnp.bitwise_and(pk, NB - 1)
                for b in range(BATCH):
                    sdesc(pk, pst, b).wait()

            nst = jnp.bitwise_and(nk, NB - 1)
            wdesc(nk, nst).start()
            for b in range(BATCH):
                gdesc(nk, nst, b).start()

        return carry

    lax.fori_loop(0, NCHUNK, chunk, 0)

    for k in range(max(0, NCHUNK - NB), NCHUNK):
        for b in range(BATCH):
            sdesc(k, k % NB, b).wait()


@jax.jit
def _embed(input_ids, wte, wpe):
    mesh = plsc.VectorSubcoreMesh(core_axis_name="c", subcore_axis_name="s")
    return pl.kernel(
        _body,
        out_type=jax.ShapeDtypeStruct((BATCH, SEQ, D), jnp.float32),
        mesh=mesh,
        scratch_types=[
            pltpu.VMEM((BATCH, P), jnp.int32),
            pltpu.VMEM((NB, ROWS, D), jnp.float32),
            pltpu.VMEM((NB, CP, D), jnp.float32),
            pltpu.SemaphoreType.DMA((NB,)),
            pltpu.SemaphoreType.DMA((NB,)),
            pltpu.SemaphoreType.DMA((NB,)),
            pltpu.SemaphoreType.DMA,
        ],
    )(input_ids, wte, wpe)


def kernel(input_ids, wte, wpe):
    return _embed(input_ids, wte, wpe)
